# baseline (speedup 1.0000x reference)
import functools

import jax
import jax.numpy as jnp
from jax import lax
from jax.experimental import pallas as pl
from jax.experimental.pallas import tpu as pltpu
from jax.experimental.pallas import tpu_sc as plsc

_WINDOW = 128
_REPLICAS = 256

_TC_BLK = 4096
_VPAD = 32

_N_SC = 53248


@functools.partial(jax.jit, static_argnames=("n", "n_sc", "embed"))
def _sc_gather_rows(table_rep, idx_sc, n, n_sc, embed):
    mesh = plsc.VectorSubcoreMesh(core_axis_name="core",
                                  subcore_axis_name="subcore")

    @functools.partial(
        pl.kernel,
        out_type=jax.ShapeDtypeStruct((n, embed), table_rep.dtype),
        mesh=mesh,
    )
    def gather_kernel(table_hbm, idx_hbm, out_hbm):
        def body(i_vmem, o_vmem):
            pltpu.sync_copy(table_hbm.at[i_vmem.at[0]], o_vmem)

        pltpu.emit_pipeline(
            body,
            grid=(n_sc // _WINDOW,),
            in_specs=[pl.BlockSpec((1, _WINDOW), index_map=lambda i: (0, i))],
            out_specs=[pl.BlockSpec((_WINDOW, embed),
                                    index_map=lambda i: (i, 0))],
            core_axis_name=("core", "subcore"),
            dimension_semantics=(pltpu.PARALLEL,),
        )(idx_hbm, out_hbm)

    return gather_kernel(table_rep, idx_sc)


def _tc_onehot_kernel(idx_ref, w_ref, buf_ref, out_ref):
    del buf_ref
    idx = idx_ref[0, 0, :]
    k_iota = lax.broadcasted_iota(jnp.int32, (_TC_BLK, 2 * _VPAD), 1)
    onehot = (jnp.bitwise_and(k_iota, _VPAD - 1) == idx[:, None])
    out_ref[...] = jnp.dot(onehot.astype(jnp.bfloat16), w_ref[...],
                           preferred_element_type=jnp.float32)


@functools.partial(jax.jit, static_argnames=("n", "n_sc", "embed"))
def _tc_onehot_rows(w_hi_lo, idx_tc, buf, n, n_sc, embed):
    nblk = (n - n_sc) // _TC_BLK
    blk0 = n_sc // _TC_BLK
    idx3 = idx_tc.reshape(nblk, 1, _TC_BLK)
    return pl.pallas_call(
        _tc_onehot_kernel,
        grid=(nblk,),
        in_specs=[
            pl.BlockSpec((1, 1, _TC_BLK), lambda i: (i, 0, 0)),
            pl.BlockSpec((2 * _VPAD, embed), lambda i: (0, 0)),
            pl.BlockSpec(memory_space=pl.ANY),
        ],
        out_specs=pl.BlockSpec((_TC_BLK, embed), lambda i: (i + blk0, 0)),
        out_shape=jax.ShapeDtypeStruct((n, embed), jnp.float32),
        input_output_aliases={2: 0},
    )(idx3, w_hi_lo, buf)


def _trunc_bf16(x):
    u = lax.bitcast_convert_type(x, jnp.uint32)
    head_f = lax.bitcast_convert_type(
        jnp.bitwise_and(u, jnp.uint32(0xFFFF0000)), jnp.float32)
    head_bf = lax.bitcast_convert_type(
        (u >> 16).astype(jnp.uint16), jnp.bfloat16)
    return head_bf, x - head_f


def _make_hi_lo(table, vocab, embed):
    tpad = jnp.zeros((_VPAD, embed), table.dtype).at[:vocab].set(table)
    hi_bf, resid = _trunc_bf16(tpad)
    lo_bf, _ = _trunc_bf16(resid)
    return jnp.concatenate([hi_bf, lo_bf], axis=0)


def kernel(seq_input, table):
    batch, seq = seq_input.shape
    vocab, embed = table.shape
    n = batch * seq
    idx_flat = seq_input.reshape(n).astype(jnp.int32)
    n_sc = _N_SC

    table_rep = jnp.tile(table, (_REPLICAS, 1))
    nblk_sc = n_sc // _WINDOW
    block_off = (jnp.arange(nblk_sc, dtype=jnp.int32) % _REPLICAS) * vocab
    idx_sc = (idx_flat[:n_sc].reshape(nblk_sc, _WINDOW)
              + block_off[:, None]).reshape(1, n_sc)

    w_hi_lo = _make_hi_lo(table, vocab, embed)

    buf = _sc_gather_rows(table_rep, idx_sc, n, n_sc, embed)
    out = _tc_onehot_rows(w_hi_lo, idx_flat[n_sc:], buf, n, n_sc, embed)
    return out.reshape(batch, seq, embed)


# speedup vs baseline: 1.0471x; 1.0471x over previous
import functools

import jax
import jax.numpy as jnp
from jax import lax
from jax.experimental import pallas as pl
from jax.experimental.pallas import tpu as pltpu
from jax.experimental.pallas import tpu_sc as plsc

_WINDOW = 128
_REPLICAS = 64

_TC_BLK = 4096
_VPAD = 32

_N_SC = 53248


@functools.partial(jax.jit, static_argnames=("n", "n_sc", "embed"))
def _sc_gather_rows(table_rep, idx_sc, n, n_sc, embed):
    mesh = plsc.VectorSubcoreMesh(core_axis_name="core",
                                  subcore_axis_name="subcore")

    @functools.partial(
        pl.kernel,
        out_type=jax.ShapeDtypeStruct((n, embed), table_rep.dtype),
        mesh=mesh,
    )
    def gather_kernel(table_hbm, idx_hbm, out_hbm):
        def body(i_vmem, o_vmem):
            pltpu.sync_copy(table_hbm.at[i_vmem.at[0]], o_vmem)

        pltpu.emit_pipeline(
            body,
            grid=(n_sc // _WINDOW,),
            in_specs=[pl.BlockSpec((1, _WINDOW), index_map=lambda i: (0, i))],
            out_specs=[pl.BlockSpec((_WINDOW, embed),
                                    index_map=lambda i: (i, 0))],
            core_axis_name=("core", "subcore"),
            dimension_semantics=(pltpu.PARALLEL,),
        )(idx_hbm, out_hbm)

    return gather_kernel(table_rep, idx_sc)


def _tc_onehot_kernel(idx_ref, w_ref, buf_ref, out_ref):
    del buf_ref
    idx = idx_ref[0, 0, :]
    k_iota = lax.broadcasted_iota(jnp.int32, (_TC_BLK, 2 * _VPAD), 1)
    onehot = (jnp.bitwise_and(k_iota, _VPAD - 1) == idx[:, None])
    out_ref[...] = jnp.dot(onehot.astype(jnp.bfloat16), w_ref[...],
                           preferred_element_type=jnp.float32)


@functools.partial(jax.jit, static_argnames=("n", "n_sc", "embed"))
def _tc_onehot_rows(w_hi_lo, idx_tc, buf, n, n_sc, embed):
    nblk = (n - n_sc) // _TC_BLK
    blk0 = n_sc // _TC_BLK
    idx3 = idx_tc.reshape(nblk, 1, _TC_BLK)
    return pl.pallas_call(
        _tc_onehot_kernel,
        grid=(nblk,),
        in_specs=[
            pl.BlockSpec((1, 1, _TC_BLK), lambda i: (i, 0, 0)),
            pl.BlockSpec((2 * _VPAD, embed), lambda i: (0, 0)),
            pl.BlockSpec(memory_space=pl.ANY),
        ],
        out_specs=pl.BlockSpec((_TC_BLK, embed), lambda i: (i + blk0, 0)),
        out_shape=jax.ShapeDtypeStruct((n, embed), jnp.float32),
        input_output_aliases={2: 0},
    )(idx3, w_hi_lo, buf)


def _trunc_bf16(x):
    u = lax.bitcast_convert_type(x, jnp.uint32)
    head_f = lax.bitcast_convert_type(
        jnp.bitwise_and(u, jnp.uint32(0xFFFF0000)), jnp.float32)
    head_bf = lax.bitcast_convert_type(
        (u >> 16).astype(jnp.uint16), jnp.bfloat16)
    return head_bf, x - head_f


def _make_hi_lo(table, vocab, embed):
    tpad = jnp.zeros((_VPAD, embed), table.dtype).at[:vocab].set(table)
    hi_bf, resid = _trunc_bf16(tpad)
    lo_bf, _ = _trunc_bf16(resid)
    return jnp.concatenate([hi_bf, lo_bf], axis=0)


def kernel(seq_input, table):
    batch, seq = seq_input.shape
    vocab, embed = table.shape
    n = batch * seq
    idx_flat = seq_input.reshape(n).astype(jnp.int32)
    n_sc = _N_SC

    table_rep = jnp.tile(table, (_REPLICAS, 1))
    nblk_sc = n_sc // _WINDOW
    block_off = (jnp.arange(nblk_sc, dtype=jnp.int32) % _REPLICAS) * vocab
    idx_sc = (idx_flat[:n_sc].reshape(nblk_sc, _WINDOW)
              + block_off[:, None]).reshape(1, n_sc)

    w_hi_lo = _make_hi_lo(table, vocab, embed)

    buf = _sc_gather_rows(table_rep, idx_sc, n, n_sc, embed)
    out = _tc_onehot_rows(w_hi_lo, idx_flat[n_sc:], buf, n, n_sc, embed)
    return out.reshape(batch, seq, embed)
